# Initial kernel scaffold; baseline (speedup 1.0000x reference)
import jax
import jax.numpy as jnp
from jax.experimental import pallas as pl


def kernel(z_e, codebook):
    raise NotImplementedError("write your pallas kernel here")


# speedup vs baseline: 2.0194x; 2.0194x over previous
import functools

import jax
import jax.numpy as jnp
from jax import lax
from jax.experimental import pallas as pl
from jax.experimental.pallas import tpu as pltpu
from jax.experimental.pallas import tpu_sc as plsc

_N_CODES = 8192
_DIM = 256
_N_TOK = 2048
_CODE_BLK = 512
_N_CODE_BLK = _N_CODES // _CODE_BLK
_TOK_BLK = 256
_N_TOK_BLK = _N_TOK // _TOK_BLK
_COMMIT = 0.25


def _argmin_body(z_ref, zs_ref, cs_ref, cb_ref, idx_ref, min_ref):
    j = pl.program_id(0)
    mm = lax.dot_general(
        z_ref[...], cb_ref[...], (((1,), (1,)), ((), ())),
        preferred_element_type=jnp.float32)
    c_blk = cs_ref[0, pl.ds(j * _CODE_BLK, _CODE_BLK)]
    d = (zs_ref[...] + c_blk) - 2.0 * mm
    rowmin = jnp.min(d, axis=1, keepdims=True)
    cols = lax.broadcasted_iota(jnp.int32, d.shape, 1)
    local = jnp.min(jnp.where(d == rowmin, cols, jnp.int32(1 << 30)),
                    axis=1, keepdims=True)
    gidx = local + j * _CODE_BLK

    @pl.when(j == 0)
    def _init():
        min_ref[...] = rowmin
        idx_ref[...] = gidx

    @pl.when(j > 0)
    def _update():
        better = rowmin < min_ref[...]
        idx_ref[...] = jnp.where(better, gidx, idx_ref[...])
        min_ref[...] = jnp.where(better, rowmin, min_ref[...])


def _argmin_call():
    def cb_map(j):
        return (j, 0)

    return pl.pallas_call(
        _argmin_body,
        grid=(_N_CODE_BLK,),
        in_specs=[
            pl.BlockSpec((_N_TOK, _DIM), lambda j: (0, 0)),
            pl.BlockSpec((_N_TOK, 1), lambda j: (0, 0)),
            pl.BlockSpec((1, _N_CODES), lambda j: (0, 0)),
            pl.BlockSpec((_CODE_BLK, _DIM), cb_map),
        ],
        out_specs=pl.BlockSpec((_N_TOK, 1), lambda j: (0, 0)),
        out_shape=jax.ShapeDtypeStruct((_N_TOK, 1), jnp.int32),
        scratch_shapes=[pltpu.VMEM((_N_TOK, 1), jnp.float32)],
        compiler_params=pltpu.CompilerParams(
            dimension_semantics=("arbitrary",)),
    )


def _make_sc_gather():
    info = plsc.get_sparse_core_info()
    nc, ns = info.num_cores, info.num_subcores
    nw = nc * ns
    b_per_w = _N_TOK // nw
    mesh = plsc.VectorSubcoreMesh(core_axis_name="c", subcore_axis_name="s")

    @functools.partial(
        pl.kernel, mesh=mesh,
        out_type=jax.ShapeDtypeStruct((_N_TOK, _DIM), jnp.float32),
        scratch_types=[
            pltpu.VMEM((b_per_w,), jnp.int32),
            pltpu.VMEM((b_per_w, _DIM), jnp.float32),
            pltpu.SemaphoreType.DMA,
        ],
    )
    def gather_k(table_hbm, idx_hbm, out_hbm, idx_v, rows_v, sem):
        wid = lax.axis_index("s") * nc + lax.axis_index("c")
        base = wid * b_per_w
        pltpu.sync_copy(idx_hbm.at[pl.ds(base, b_per_w)], idx_v)
        pltpu.async_copy(table_hbm.at[idx_v], rows_v, sem).wait()
        pltpu.sync_copy(rows_v, out_hbm.at[pl.ds(base, b_per_w)])

    return gather_k


def _finalize_body(z_ref, q_ref, idxb_ref, idxall_ref,
                   qout_ref, loss_ref, perp_ref):
    t = pl.program_id(0)
    z = z_ref[...]
    q = q_ref[...]
    zn = z / jnp.maximum(
        jnp.sqrt(jnp.sum(z * z, axis=1, keepdims=True)), 1e-12)
    qn = q / jnp.maximum(
        jnp.sqrt(jnp.sum(q * q, axis=1, keepdims=True)), 1e-12)
    v = zn - qn
    vnorm = jnp.sqrt(jnp.sum(v * v, axis=1, keepdims=True))
    mask = (vnorm > 1e-5).astype(jnp.float32)
    v = mask * v / (vnorm + 1e-8) + (1.0 - mask) * v
    vz = jnp.sum(v * z, axis=1, keepdims=True)
    rot = z - 2.0 * v * vz
    qout_ref[...] = rot
    loss_part = jnp.sum((rot - z) ** 2)

    myidx = idxb_ref[0, 0, :]
    counts = jnp.zeros((_TOK_BLK,), jnp.float32)
    for r in range(_N_TOK_BLK):
        row = idxall_ref[r, 0, :]
        eq = (myidx[:, None] == row[None, :]).astype(jnp.float32)
        counts = counts + jnp.sum(eq, axis=1)
    ent_part = jnp.sum(
        jnp.log(counts * (1.0 / _N_TOK) + 1e-10)) * (1.0 / _N_TOK)

    @pl.when(t == 0)
    def _init():
        loss_ref[0, 0] = 0.0
        perp_ref[0, 0] = 0.0

    loss_ref[0, 0] += loss_part
    perp_ref[0, 0] += ent_part

    @pl.when(t == _N_TOK_BLK - 1)
    def _fin():
        loss_ref[0, 0] = loss_ref[0, 0] * (
            (1.0 + _COMMIT) / (_N_TOK * _DIM))
        perp_ref[0, 0] = jnp.exp(-perp_ref[0, 0])


def _finalize_call(z, quant, idx3):
    return pl.pallas_call(
        _finalize_body,
        grid=(_N_TOK_BLK,),
        in_specs=[
            pl.BlockSpec((_TOK_BLK, _DIM), lambda t: (t, 0)),
            pl.BlockSpec((_TOK_BLK, _DIM), lambda t: (t, 0)),
            pl.BlockSpec((1, 1, _TOK_BLK), lambda t: (t, 0, 0)),
            pl.BlockSpec((_N_TOK_BLK, 1, _TOK_BLK), lambda t: (0, 0, 0)),
        ],
        out_specs=[
            pl.BlockSpec((_TOK_BLK, _DIM), lambda t: (t, 0)),
            pl.BlockSpec((1, 1), lambda t: (0, 0), memory_space=pltpu.SMEM),
            pl.BlockSpec((1, 1), lambda t: (0, 0), memory_space=pltpu.SMEM),
        ],
        out_shape=[
            jax.ShapeDtypeStruct((_N_TOK, _DIM), jnp.float32),
            jax.ShapeDtypeStruct((1, 1), jnp.float32),
            jax.ShapeDtypeStruct((1, 1), jnp.float32),
        ],
        compiler_params=pltpu.CompilerParams(
            dimension_semantics=("arbitrary",)),
    )(z, quant, idx3, idx3)


def kernel(z_e, codebook):
    z = z_e.reshape(-1, _DIM)
    zs = jnp.sum(z ** 2, axis=1, keepdims=True)
    cs = jnp.sum(codebook ** 2, axis=1).reshape(1, _N_CODES)
    idx2d = _argmin_call()(z, zs, cs, codebook)
    idx = idx2d.reshape(_N_TOK)
    quant = _make_sc_gather()(codebook, idx)
    idx3 = idx.reshape(_N_TOK_BLK, 1, _TOK_BLK)
    q_out, loss, perp = _finalize_call(z, quant, idx3)
    return (q_out.reshape(z_e.shape), loss.reshape(()),
            perp.reshape(()), idx)
